# baseline (speedup 1.0000x reference)
import functools

import jax
import jax.numpy as jnp
from jax import lax
from jax.experimental import pallas as pl
from jax.experimental.pallas import tpu as pltpu
from jax.experimental.pallas import tpu_sc as plsc

_SEQ_LEN = 8192
_OUT_DIM = 1024
_NC = 2
_NS = 16
_NW = _NC * _NS
_ROWS_PER_W = _SEQ_LEN // _NW


def _make_sc_copy():
    mesh = plsc.VectorSubcoreMesh(core_axis_name="c", subcore_axis_name="s")

    @functools.partial(
        pl.kernel,
        mesh=mesh,
        out_type=jax.ShapeDtypeStruct((_SEQ_LEN, _OUT_DIM), jnp.float32),
    )
    def copy_k(table_hbm, out_hbm):
        wid = lax.axis_index("s") * _NC + lax.axis_index("c")
        base = wid * _ROWS_PER_W
        pltpu.sync_copy(
            table_hbm.at[pl.ds(base, _ROWS_PER_W)],
            out_hbm.at[pl.ds(base, _ROWS_PER_W)],
        )

    return copy_k


_sc_copy = _make_sc_copy()


@jax.jit
def kernel(inputs, pos_table):
    del inputs
    return _sc_copy(pos_table)


# speedup vs baseline: 24.6617x; 24.6617x over previous
import functools

import jax
import jax.numpy as jnp
from jax import lax
from jax.experimental import pallas as pl
from jax.experimental.pallas import tpu as pltpu
from jax.experimental.pallas import tpu_sc as plsc

_SEQ_LEN = 8192
_OUT_DIM = 1024
_NC = 2
_NS = 16
_NW = _NC * _NS
_ROWS_PER_W = _SEQ_LEN // _NW


_CHUNK = 32
_NCHUNKS = _ROWS_PER_W // _CHUNK


def _make_sc_copy():
    mesh = plsc.VectorSubcoreMesh(core_axis_name="c", subcore_axis_name="s")

    @functools.partial(
        pl.kernel,
        mesh=mesh,
        out_type=jax.ShapeDtypeStruct((_SEQ_LEN, _OUT_DIM), jnp.float32),
        scratch_types=[
            pltpu.VMEM((2, _CHUNK, _OUT_DIM), jnp.float32),
            pltpu.SemaphoreType.DMA,
            pltpu.SemaphoreType.DMA,
        ],
    )
    def copy_k(table_hbm, out_hbm, buf, gsem, ssem):
        wid = lax.axis_index("s") * _NC + lax.axis_index("c")
        base = wid * _ROWS_PER_W

        def gather(i):
            return pltpu.async_copy(
                table_hbm.at[pl.ds(base + i * _CHUNK, _CHUNK)],
                buf.at[i % 2],
                gsem,
            )

        def scatter(i):
            return pltpu.async_copy(
                buf.at[i % 2],
                out_hbm.at[pl.ds(base + i * _CHUNK, _CHUNK)],
                ssem,
            )

        gathers = [None] * _NCHUNKS
        scatters = [None] * _NCHUNKS
        gathers[0] = gather(0)
        for i in range(_NCHUNKS):
            if i >= 1:
                scatters[i - 1].wait()
            if i + 1 < _NCHUNKS:
                gathers[i + 1] = gather(i + 1)
            gathers[i].wait()
            scatters[i] = scatter(i)
        scatters[_NCHUNKS - 1].wait()

    return copy_k


_sc_copy = _make_sc_copy()


@jax.jit
def kernel(inputs, pos_table):
    del inputs
    return _sc_copy(pos_table)


# speedup vs baseline: 24.8265x; 1.0067x over previous
import functools

import jax
import jax.numpy as jnp
from jax import lax
from jax.experimental import pallas as pl
from jax.experimental.pallas import tpu as pltpu
from jax.experimental.pallas import tpu_sc as plsc

_SEQ_LEN = 8192
_OUT_DIM = 1024
_NC = 2
_NS = 16
_NW = _NC * _NS
_ROWS_PER_W = _SEQ_LEN // _NW


_CHUNK = 32
_NCHUNKS = _ROWS_PER_W // _CHUNK
_NBUF = 3


def _make_sc_copy():
    mesh = plsc.VectorSubcoreMesh(core_axis_name="c", subcore_axis_name="s")

    @functools.partial(
        pl.kernel,
        mesh=mesh,
        out_type=jax.ShapeDtypeStruct((_SEQ_LEN, _OUT_DIM), jnp.float32),
        scratch_types=[
            pltpu.VMEM((_NBUF, _CHUNK, _OUT_DIM), jnp.float32),
            pltpu.SemaphoreType.DMA,
            pltpu.SemaphoreType.DMA,
        ],
    )
    def copy_k(table_hbm, out_hbm, buf, gsem, ssem):
        wid = lax.axis_index("s") * _NC + lax.axis_index("c")
        base = wid * _ROWS_PER_W

        def gather(i):
            return pltpu.async_copy(
                table_hbm.at[pl.ds(base + i * _CHUNK, _CHUNK)],
                buf.at[i % _NBUF],
                gsem,
            )

        def scatter(i):
            return pltpu.async_copy(
                buf.at[i % _NBUF],
                out_hbm.at[pl.ds(base + i * _CHUNK, _CHUNK)],
                ssem,
            )

        gathers = [None] * _NCHUNKS
        scatters = [None] * _NCHUNKS
        for i in range(_NCHUNKS + 1):
            if i < _NCHUNKS:
                if i >= _NBUF:
                    scatters[i - _NBUF].wait()
                gathers[i] = gather(i)
            if i >= 1:
                gathers[i - 1].wait()
                scatters[i - 1] = scatter(i - 1)
        scatters[_NCHUNKS - 1].wait()

    return copy_k


_sc_copy = _make_sc_copy()


@jax.jit
def kernel(inputs, pos_table):
    del inputs
    return _sc_copy(pos_table)
